# baseline (speedup 1.0000x reference)
import functools

import jax
import jax.numpy as jnp
from jax import lax
from jax.experimental import pallas as pl
from jax.experimental.pallas import tpu as pltpu
from jax.experimental.pallas import tpu_sc as plsc

_WN = 10
_K = 8192
_PCA = 4096
_D = 512
_N = 10240
_G = _N // _WN

_KB = 512
_GB = 16


def _mapped_body(cb_ref, w_ref, b_ref, mapped_ref, msq_ref):
    m = lax.dot_general(cb_ref[...], w_ref[...], (((1,), (1,)), ((), ())),
                        preferred_element_type=jnp.float32)
    m = m + b_ref[...]
    mapped_ref[...] = m
    msq_ref[...] = jnp.sum(m * m, axis=1, keepdims=True)


def _mapped_call(cb, w, b2):
    return pl.pallas_call(
        _mapped_body,
        grid=(_K // _KB,),
        in_specs=[
            pl.BlockSpec((_KB, _PCA), lambda j: (j, 0)),
            pl.BlockSpec((_D, _PCA), lambda j: (0, 0)),
            pl.BlockSpec((1, _D), lambda j: (0, 0)),
        ],
        out_specs=[
            pl.BlockSpec((_KB, _D), lambda j: (j, 0)),
            pl.BlockSpec((_KB, 1), lambda j: (j, 0)),
        ],
        out_shape=[
            jax.ShapeDtypeStruct((_K, _D), jnp.float32),
            jax.ShapeDtypeStruct((_K, 1), jnp.float32),
        ],
    )(cb, w, b2)


def _assign_body(zw_ref, zsq_ref, mt_ref, msq_ref, idx_ref):
    zb = zw_ref[...]
    z2 = zb.reshape(_WN * _GB, _D)
    dot = lax.dot_general(z2, mt_ref[...], (((1,), (0,)), ((), ())),
                          preferred_element_type=jnp.float32)
    dot3 = dot.reshape(_WN, _GB, _K)
    zsq = zsq_ref[...]
    sums = zsq + msq_ref[...].reshape(1, 1, _K)
    dist = sums - 2.0 * dot3
    col = lax.broadcasted_iota(jnp.int32, (_GB, _K), 1)
    masked = jnp.zeros((_GB, _K), jnp.bool_)
    cols = []
    for i in range(_WN):
        di = jnp.where(masked, jnp.inf, dist[i])
        mval = jnp.min(di, axis=1, keepdims=True)
        cand = jnp.where(di == mval, col, jnp.int32(_K))
        idx_i = jnp.min(cand, axis=1)
        cols.append(idx_i)
        masked = jnp.logical_or(masked, col == idx_i[:, None])
    idx_ref[...] = jnp.stack(cols, axis=1)


def _assign_call(zw, zsq3, mt, msq_row):
    return pl.pallas_call(
        _assign_body,
        grid=(_G // _GB,),
        in_specs=[
            pl.BlockSpec((_WN, _GB, _D), lambda j: (0, j, 0)),
            pl.BlockSpec((_WN, _GB, 1), lambda j: (0, j, 0)),
            pl.BlockSpec((_D, _K), lambda j: (0, 0)),
            pl.BlockSpec((1, _K), lambda j: (0, 0)),
        ],
        out_specs=pl.BlockSpec((_GB, _WN), lambda j: (j, 0)),
        out_shape=jax.ShapeDtypeStruct((_G, _WN), jnp.int32),
    )(zw, zsq3, mt, msq_row)


_NW_SC = 32
_CH = 80
_NCH = _N // (_NW_SC * _CH)


def _sc_gather_st_loss(mapped, idx2, z_e2):
    mesh = plsc.VectorSubcoreMesh(core_axis_name="c", subcore_axis_name="s")

    @functools.partial(
        pl.kernel,
        out_type=(jax.ShapeDtypeStruct((_N, _D), jnp.float32),
                  jax.ShapeDtypeStruct((_NW_SC, 16), jnp.float32)),
        mesh=mesh,
        scratch_types=[
            pltpu.VMEM((_CH,), jnp.int32),
            pltpu.VMEM((_CH, _D), jnp.float32),
            pltpu.VMEM((_CH, _D), jnp.float32),
            pltpu.VMEM((16,), jnp.float32),
            pltpu.SemaphoreType.DMA,
        ],
    )
    def body(mapped_hbm, idx_hbm, z_hbm, out_hbm, lp_hbm,
             idx_v, rows_v, z_v, acc_v, sem):
        wid = lax.axis_index("s") * 2 + lax.axis_index("c")
        zero = jnp.zeros((16,), jnp.float32)
        accs = (zero, zero, zero, zero, zero, zero, zero, zero)
        for ci in range(_NCH):
            chunk = wid * _NCH + ci
            base = chunk * _CH
            pltpu.sync_copy(idx_hbm.at[chunk], idx_v)
            pltpu.async_copy(mapped_hbm.at[idx_v], rows_v, sem).wait()
            pltpu.sync_copy(z_hbm.at[pl.ds(base, _CH)], z_v)

            def row_body(r, carry):
                a = list(carry)
                for cc in range(_D // 16):
                    sl = pl.ds(cc * 16, 16)
                    q = rows_v[r, sl]
                    z = z_v[r, sl]
                    t = q - z
                    rows_v[r, sl] = z + t
                    a[cc % 8] = a[cc % 8] + t * t
                return tuple(a)

            accs = lax.fori_loop(0, _CH, row_body, accs)
            pltpu.sync_copy(rows_v, out_hbm.at[pl.ds(base, _CH)])
        a = list(accs)
        acc_v[...] = ((a[0] + a[1]) + (a[2] + a[3])) + ((a[4] + a[5]) + (a[6] + a[7]))
        pltpu.sync_copy(acc_v, lp_hbm.at[wid])

    return body(mapped, idx2, z_e2)


def kernel(z_e, codebook_pca, W, b):
    mapped, msq_col = _mapped_call(codebook_pca, W, b[None, :])
    mt = mapped.T
    msq_row = msq_col.reshape(1, _K)
    zw = z_e.reshape(_G, _WN, _D).transpose(1, 0, 2)
    zsq3 = jnp.sum(z_e ** 2, axis=1).reshape(_G, _WN).T[..., None]
    idxs = _assign_call(zw, zsq3, mt, msq_row)
    idx2 = idxs.reshape(_NW_SC * _NCH, _CH)
    z_q_st, lp = _sc_gather_st_loss(mapped, idx2, z_e)
    vq = jnp.sum(lp) * (1.0 / (_N * _D))
    loss = 0.75 * vq + 0.25 * vq
    return (z_q_st, loss)


# speedup vs baseline: 1.0567x; 1.0567x over previous
import functools

import jax
import jax.numpy as jnp
from jax import lax
from jax.experimental import pallas as pl
from jax.experimental.pallas import tpu as pltpu
from jax.experimental.pallas import tpu_sc as plsc

_WN = 10
_K = 8192
_PCA = 4096
_D = 512
_N = 10240
_G = _N // _WN

_KB = 512
_GB = 16


def _mapped_body(cb_ref, w_ref, b_ref, mapped_ref, msq_ref):
    m = lax.dot_general(cb_ref[...], w_ref[...], (((1,), (1,)), ((), ())),
                        preferred_element_type=jnp.float32)
    m = m + b_ref[...]
    mapped_ref[...] = m
    msq_ref[...] = jnp.sum(m * m, axis=1, keepdims=True)


def _mapped_call(cb, w, b2):
    return pl.pallas_call(
        _mapped_body,
        grid=(_K // _KB,),
        in_specs=[
            pl.BlockSpec((_KB, _PCA), lambda j: (j, 0)),
            pl.BlockSpec((_D, _PCA), lambda j: (0, 0)),
            pl.BlockSpec((1, _D), lambda j: (0, 0)),
        ],
        out_specs=[
            pl.BlockSpec((_KB, _D), lambda j: (j, 0)),
            pl.BlockSpec((_KB, 1), lambda j: (j, 0)),
        ],
        out_shape=[
            jax.ShapeDtypeStruct((_K, _D), jnp.float32),
            jax.ShapeDtypeStruct((_K, 1), jnp.float32),
        ],
    )(cb, w, b2)


_NB = _G // _GB


def _dist_into(zw_ref, zsq_ref, mt_ref, msq_ref, buf_ref):
    zb = zw_ref[...]
    z2 = zb.reshape(_WN * _GB, _D)
    dot = lax.dot_general(z2, mt_ref[...], (((1,), (0,)), ((), ())),
                          preferred_element_type=jnp.float32)
    dot3 = dot.reshape(_WN, _GB, _K)
    zsq = zsq_ref[...]
    sums = zsq + msq_ref[...].reshape(1, 1, _K)
    buf_ref[...] = sums - 2.0 * dot3


def _argmin_from(buf_ref, idx_ref):
    dist = buf_ref[...]
    col = lax.broadcasted_iota(jnp.int32, (_GB, _K), 1)
    masked = jnp.zeros((_GB, _K), jnp.bool_)
    cols = []
    for i in range(_WN):
        di = jnp.where(masked, jnp.inf, dist[i])
        mval = jnp.min(di, axis=1, keepdims=True)
        cand = jnp.where(di == mval, col, jnp.int32(_K))
        idx_i = jnp.min(cand, axis=1)
        cols.append(idx_i)
        masked = jnp.logical_or(masked, col == idx_i[:, None])
    idx_ref[...] = jnp.stack(cols, axis=1)


def _assign_body(zw_ref, zsq_ref, mt_ref, msq_ref, idx_ref, buf_a, buf_b):
    j = pl.program_id(0)
    even = lax.rem(j, 2) == 0

    @pl.when(jnp.logical_and(j < _NB, even))
    def _():
        _dist_into(zw_ref, zsq_ref, mt_ref, msq_ref, buf_a)

    @pl.when(jnp.logical_and(j < _NB, jnp.logical_not(even)))
    def _():
        _dist_into(zw_ref, zsq_ref, mt_ref, msq_ref, buf_b)

    @pl.when(jnp.logical_and(j > 0, even))
    def _():
        _argmin_from(buf_b, idx_ref)

    @pl.when(jnp.logical_and(j > 0, jnp.logical_not(even)))
    def _():
        _argmin_from(buf_a, idx_ref)


def _assign_call(zw, zsq3, mt, msq_row):
    return pl.pallas_call(
        _assign_body,
        grid=(_NB + 1,),
        in_specs=[
            pl.BlockSpec((_WN, _GB, _D), lambda j: (0, jnp.minimum(j, _NB - 1), 0)),
            pl.BlockSpec((_WN, _GB, 1), lambda j: (0, jnp.minimum(j, _NB - 1), 0)),
            pl.BlockSpec((_D, _K), lambda j: (0, 0)),
            pl.BlockSpec((1, _K), lambda j: (0, 0)),
        ],
        out_specs=pl.BlockSpec((_GB, _WN), lambda j: (jnp.maximum(j - 1, 0), 0)),
        out_shape=jax.ShapeDtypeStruct((_G, _WN), jnp.int32),
        scratch_shapes=[
            pltpu.VMEM((_WN, _GB, _K), jnp.float32),
            pltpu.VMEM((_WN, _GB, _K), jnp.float32),
        ],
    )(zw, zsq3, mt, msq_row)


_NW_SC = 32
_CH = 80
_NCH = _N // (_NW_SC * _CH)


def _sc_gather_st_loss(mapped, idx2, z_e2):
    mesh = plsc.VectorSubcoreMesh(core_axis_name="c", subcore_axis_name="s")

    @functools.partial(
        pl.kernel,
        out_type=(jax.ShapeDtypeStruct((_N, _D), jnp.float32),
                  jax.ShapeDtypeStruct((_NW_SC, 16), jnp.float32)),
        mesh=mesh,
        scratch_types=[
            pltpu.VMEM((_CH,), jnp.int32),
            pltpu.VMEM((_CH, _D), jnp.float32),
            pltpu.VMEM((_CH, _D), jnp.float32),
            pltpu.VMEM((16,), jnp.float32),
            pltpu.SemaphoreType.DMA,
        ],
    )
    def body(mapped_hbm, idx_hbm, z_hbm, out_hbm, lp_hbm,
             idx_v, rows_v, z_v, acc_v, sem):
        wid = lax.axis_index("s") * 2 + lax.axis_index("c")
        zero = jnp.zeros((16,), jnp.float32)
        accs = (zero, zero, zero, zero, zero, zero, zero, zero)
        for ci in range(_NCH):
            chunk = wid * _NCH + ci
            base = chunk * _CH
            pltpu.sync_copy(idx_hbm.at[chunk], idx_v)
            pltpu.async_copy(mapped_hbm.at[idx_v], rows_v, sem).wait()
            pltpu.sync_copy(z_hbm.at[pl.ds(base, _CH)], z_v)

            def row_body(r, carry):
                a = list(carry)
                for cc in range(_D // 16):
                    sl = pl.ds(cc * 16, 16)
                    q = rows_v[r, sl]
                    z = z_v[r, sl]
                    t = q - z
                    rows_v[r, sl] = z + t
                    a[cc % 8] = a[cc % 8] + t * t
                return tuple(a)

            accs = lax.fori_loop(0, _CH, row_body, accs)
            pltpu.sync_copy(rows_v, out_hbm.at[pl.ds(base, _CH)])
        a = list(accs)
        acc_v[...] = ((a[0] + a[1]) + (a[2] + a[3])) + ((a[4] + a[5]) + (a[6] + a[7]))
        pltpu.sync_copy(acc_v, lp_hbm.at[wid])

    return body(mapped, idx2, z_e2)


def kernel(z_e, codebook_pca, W, b):
    mapped, msq_col = _mapped_call(codebook_pca, W, b[None, :])
    mt = mapped.T
    msq_row = msq_col.reshape(1, _K)
    zw = z_e.reshape(_G, _WN, _D).transpose(1, 0, 2)
    zsq3 = jnp.sum(z_e ** 2, axis=1).reshape(_G, _WN).T[..., None]
    idxs = _assign_call(zw, zsq3, mt, msq_row)
    idx2 = idxs.reshape(_NW_SC * _NCH, _CH)
    z_q_st, lp = _sc_gather_st_loss(mapped, idx2, z_e)
    vq = jnp.sum(lp) * (1.0 / (_N * _D))
    loss = 0.75 * vq + 0.25 * vq
    return (z_q_st, loss)


# speedup vs baseline: 1.1564x; 1.0944x over previous
import functools

import jax
import jax.numpy as jnp
from jax import lax
from jax.experimental import pallas as pl
from jax.experimental.pallas import tpu as pltpu
from jax.experimental.pallas import tpu_sc as plsc

_WN = 10
_K = 8192
_PCA = 4096
_D = 512
_N = 10240
_G = _N // _WN

_KB = 512
_GB = 16


def _mapped_body(cb_ref, w_ref, b_ref, mapped_ref, msq_ref):
    m = lax.dot_general(cb_ref[...], w_ref[...], (((1,), (1,)), ((), ())),
                        preferred_element_type=jnp.float32)
    m = m + b_ref[...]
    mapped_ref[...] = m
    msq_ref[...] = jnp.sum(m * m, axis=1, keepdims=True)


def _mapped_call(cb, w, b2):
    return pl.pallas_call(
        _mapped_body,
        grid=(_K // _KB,),
        in_specs=[
            pl.BlockSpec((_KB, _PCA), lambda j: (j, 0)),
            pl.BlockSpec((_D, _PCA), lambda j: (0, 0)),
            pl.BlockSpec((1, _D), lambda j: (0, 0)),
        ],
        out_specs=[
            pl.BlockSpec((_KB, _D), lambda j: (j, 0)),
            pl.BlockSpec((_KB, 1), lambda j: (j, 0)),
        ],
        out_shape=[
            jax.ShapeDtypeStruct((_K, _D), jnp.float32),
            jax.ShapeDtypeStruct((_K, 1), jnp.float32),
        ],
    )(cb, w, b2)


_NB = _G // _GB
_GB2 = 2 * _GB


def _dist_into(zb, zsqb, mt_ref, msq_ref, buf_ref):
    z2 = zb.reshape(_WN * _GB, _D)
    dot = lax.dot_general(z2, mt_ref[...], (((1,), (0,)), ((), ())),
                          preferred_element_type=jnp.float32)
    sums = zsqb + msq_ref[...].reshape(1, 1, _K)
    buf_ref[...] = sums - dot.reshape(_WN, _GB, _K)


def _argmin_store(buf_ref, idx_ref, row0):
    dist = buf_ref[...]
    col = lax.broadcasted_iota(jnp.int32, (_GB, _K), 1)
    masked = jnp.zeros((_GB, _K), jnp.bool_)
    cols = []
    for i in range(_WN):
        di = jnp.where(masked, jnp.inf, dist[i])
        mval = jnp.min(di, axis=1, keepdims=True)
        cand = jnp.where(di == mval, col, jnp.int32(_K))
        idx_i = jnp.min(cand, axis=1)
        cols.append(idx_i)
        masked = jnp.logical_or(masked, col == idx_i[:, None])
    idx_ref[pl.ds(row0, _GB), :] = jnp.stack(cols, axis=1)


def _assign_body(zw_ref, zsq_ref, mt_ref, msq_ref, idx_ref, buf_a, buf_b):
    j = pl.program_id(0)
    zb = zw_ref[...]
    zq = zsq_ref[...]
    row_a = jnp.maximum(2 * j - 1, 0) * _GB
    row_b = jnp.minimum(2 * j, _NB - 2) * _GB
    _argmin_store(buf_a, idx_ref, row_a)
    _dist_into(zb[:, :_GB], zq[:, :_GB], mt_ref, msq_ref, buf_b)
    _argmin_store(buf_b, idx_ref, row_b)
    _dist_into(zb[:, _GB:], zq[:, _GB:], mt_ref, msq_ref, buf_a)


def _assign_call(zw, zsq3, mt, msq_row):
    nsteps = _NB // 2
    return pl.pallas_call(
        _assign_body,
        grid=(nsteps + 1,),
        in_specs=[
            pl.BlockSpec((_WN, _GB2, _D), lambda j: (0, jnp.minimum(j, _NB // 2 - 1), 0)),
            pl.BlockSpec((_WN, _GB2, 1), lambda j: (0, jnp.minimum(j, _NB // 2 - 1), 0)),
            pl.BlockSpec((_D, _K), lambda j: (0, 0)),
            pl.BlockSpec((1, _K), lambda j: (0, 0)),
        ],
        out_specs=pl.BlockSpec((_G, _WN), lambda j: (0, 0)),
        out_shape=jax.ShapeDtypeStruct((_G, _WN), jnp.int32),
        scratch_shapes=[
            pltpu.VMEM((_WN, _GB, _K), jnp.float32),
            pltpu.VMEM((_WN, _GB, _K), jnp.float32),
        ],
    )(zw, zsq3, mt, msq_row)


_NW_SC = 32
_CH = 80
_NCH = _N // (_NW_SC * _CH)


def _sc_gather_st_loss(mapped, idx2, z_e2):
    mesh = plsc.VectorSubcoreMesh(core_axis_name="c", subcore_axis_name="s")

    @functools.partial(
        pl.kernel,
        out_type=(jax.ShapeDtypeStruct((_N, _D), jnp.float32),
                  jax.ShapeDtypeStruct((_NW_SC, 16), jnp.float32)),
        mesh=mesh,
        scratch_types=[
            pltpu.VMEM((_CH,), jnp.int32),
            pltpu.VMEM((_CH, _D), jnp.float32),
            pltpu.VMEM((_CH, _D), jnp.float32),
            pltpu.VMEM((16,), jnp.float32),
            pltpu.SemaphoreType.DMA,
        ],
    )
    def body(mapped_hbm, idx_hbm, z_hbm, out_hbm, lp_hbm,
             idx_v, rows_v, z_v, acc_v, sem):
        wid = lax.axis_index("s") * 2 + lax.axis_index("c")
        zero = jnp.zeros((16,), jnp.float32)
        accs = (zero, zero, zero, zero, zero, zero, zero, zero)
        for ci in range(_NCH):
            chunk = wid * _NCH + ci
            base = chunk * _CH
            pltpu.sync_copy(idx_hbm.at[chunk], idx_v)
            pltpu.async_copy(mapped_hbm.at[idx_v], rows_v, sem).wait()
            pltpu.sync_copy(z_hbm.at[pl.ds(base, _CH)], z_v)

            def row_body(r, carry):
                a = list(carry)
                for cc in range(_D // 16):
                    sl = pl.ds(cc * 16, 16)
                    q = rows_v[r, sl]
                    z = z_v[r, sl]
                    t = q - z
                    rows_v[r, sl] = z + t
                    a[cc % 8] = a[cc % 8] + t * t
                return tuple(a)

            accs = lax.fori_loop(0, _CH, row_body, accs)
            pltpu.sync_copy(rows_v, out_hbm.at[pl.ds(base, _CH)])
        a = list(accs)
        acc_v[...] = ((a[0] + a[1]) + (a[2] + a[3])) + ((a[4] + a[5]) + (a[6] + a[7]))
        pltpu.sync_copy(acc_v, lp_hbm.at[wid])

    return body(mapped, idx2, z_e2)


def kernel(z_e, codebook_pca, W, b):
    mapped, msq_col = _mapped_call(codebook_pca, W, b[None, :])
    mt = (mapped + mapped).T
    msq_row = msq_col.reshape(1, _K)
    zw = z_e.reshape(_G, _WN, _D).transpose(1, 0, 2)
    zsq3 = jnp.sum(z_e ** 2, axis=1).reshape(_G, _WN).T[..., None]
    idxs = _assign_call(zw, zsq3, mt, msq_row)
    idx2 = idxs.reshape(_NW_SC * _NCH, _CH)
    z_q_st, lp = _sc_gather_st_loss(mapped, idx2, z_e)
    vq = jnp.sum(lp) * (1.0 / (_N * _D))
    loss = 0.75 * vq + 0.25 * vq
    return (z_q_st, loss)


# speedup vs baseline: 1.2256x; 1.0598x over previous
import functools

import jax
import jax.numpy as jnp
from jax import lax
from jax.experimental import pallas as pl
from jax.experimental.pallas import tpu as pltpu
from jax.experimental.pallas import tpu_sc as plsc

_WN = 10
_K = 8192
_PCA = 4096
_D = 512
_N = 10240
_G = _N // _WN

_KB = 512
_GB = 16


def _mapped_body(cb_ref, w_ref, b_ref, mapped_ref, msq_ref, mt_ref):
    m = lax.dot_general(cb_ref[...], w_ref[...], (((1,), (1,)), ((), ())),
                        preferred_element_type=jnp.float32)
    m = m + b_ref[...]
    mapped_ref[...] = m
    msq_ref[...] = jnp.sum(m * m, axis=1, keepdims=True)
    mt_ref[...] = (m + m).T


def _mapped_call(cb, w, b2):
    return pl.pallas_call(
        _mapped_body,
        grid=(_K // _KB,),
        in_specs=[
            pl.BlockSpec((_KB, _PCA), lambda j: (j, 0)),
            pl.BlockSpec((_D, _PCA), lambda j: (0, 0)),
            pl.BlockSpec((1, _D), lambda j: (0, 0)),
        ],
        out_specs=[
            pl.BlockSpec((_KB, _D), lambda j: (j, 0)),
            pl.BlockSpec((_KB, 1), lambda j: (j, 0)),
            pl.BlockSpec((_D, _KB), lambda j: (0, j)),
        ],
        out_shape=[
            jax.ShapeDtypeStruct((_K, _D), jnp.float32),
            jax.ShapeDtypeStruct((_K, 1), jnp.float32),
            jax.ShapeDtypeStruct((_D, _K), jnp.float32),
        ],
    )(cb, w, b2)


_NB = _G // _GB
_GB2 = 2 * _GB


def _dist_into(zb, zsqb, mt_ref, msq_ref, buf_ref):
    z2 = zb.reshape(_WN * _GB, _D)
    dot = lax.dot_general(z2, mt_ref[...], (((1,), (0,)), ((), ())),
                          preferred_element_type=jnp.float32)
    sums = zsqb + msq_ref[...].reshape(1, 1, _K)
    buf_ref[...] = sums - dot.reshape(_WN, _GB, _K)


def _argmin_store(buf_ref, idx_ref, row0):
    dist = buf_ref[...]
    col = lax.broadcasted_iota(jnp.int32, (_GB, _K), 1)
    masked = jnp.zeros((_GB, _K), jnp.bool_)
    cols = []
    for i in range(_WN):
        di = jnp.where(masked, jnp.inf, dist[i])
        mval = jnp.min(di, axis=1, keepdims=True)
        cand = jnp.where(di == mval, col, jnp.int32(_K))
        idx_i = jnp.min(cand, axis=1)
        cols.append(idx_i)
        masked = jnp.logical_or(masked, col == idx_i[:, None])
    idx_ref[pl.ds(row0, _GB), :] = jnp.stack(cols, axis=1)


def _assign_body(zw_ref, zsq_ref, mt_ref, msq_ref, idx_ref, buf_a, buf_b):
    j = pl.program_id(0)
    zb = zw_ref[...]
    zq = zsq_ref[...]
    row_a = jnp.maximum(2 * j - 1, 0) * _GB
    row_b = jnp.minimum(2 * j, _NB - 2) * _GB
    _argmin_store(buf_a, idx_ref, row_a)
    _dist_into(zb[:, :_GB], zq[:, :_GB], mt_ref, msq_ref, buf_b)
    _argmin_store(buf_b, idx_ref, row_b)
    _dist_into(zb[:, _GB:], zq[:, _GB:], mt_ref, msq_ref, buf_a)


def _assign_call(zw, zsq3, mt, msq_row):
    nsteps = _NB // 2
    return pl.pallas_call(
        _assign_body,
        grid=(nsteps + 1,),
        in_specs=[
            pl.BlockSpec((_WN, _GB2, _D), lambda j: (0, jnp.minimum(j, _NB // 2 - 1), 0)),
            pl.BlockSpec((_WN, _GB2, 1), lambda j: (0, jnp.minimum(j, _NB // 2 - 1), 0)),
            pl.BlockSpec((_D, _K), lambda j: (0, 0)),
            pl.BlockSpec((1, _K), lambda j: (0, 0)),
        ],
        out_specs=pl.BlockSpec((_G, _WN), lambda j: (0, 0)),
        out_shape=jax.ShapeDtypeStruct((_G, _WN), jnp.int32),
        scratch_shapes=[
            pltpu.VMEM((_WN, _GB, _K), jnp.float32),
            pltpu.VMEM((_WN, _GB, _K), jnp.float32),
        ],
    )(zw, zsq3, mt, msq_row)


_NW_SC = 32
_CH = 80
_NCH = _N // (_NW_SC * _CH)


def _sc_gather_st_loss(mapped, idx2, z_e2):
    mesh = plsc.VectorSubcoreMesh(core_axis_name="c", subcore_axis_name="s")

    @functools.partial(
        pl.kernel,
        out_type=(jax.ShapeDtypeStruct((_N, _D), jnp.float32),
                  jax.ShapeDtypeStruct((_NW_SC, 16), jnp.float32)),
        mesh=mesh,
        scratch_types=[
            pltpu.VMEM((_CH,), jnp.int32),
            pltpu.VMEM((_CH, _D), jnp.float32),
            pltpu.VMEM((_CH, _D), jnp.float32),
            pltpu.VMEM((16,), jnp.float32),
            pltpu.SemaphoreType.DMA,
        ],
    )
    def body(mapped_hbm, idx_hbm, z_hbm, out_hbm, lp_hbm,
             idx_v, rows_v, z_v, acc_v, sem):
        wid = lax.axis_index("s") * 2 + lax.axis_index("c")
        zero = jnp.zeros((16,), jnp.float32)
        accs = (zero, zero, zero, zero, zero, zero, zero, zero)
        for ci in range(_NCH):
            chunk = wid * _NCH + ci
            base = chunk * _CH
            pltpu.sync_copy(idx_hbm.at[chunk], idx_v)
            pltpu.async_copy(mapped_hbm.at[idx_v], rows_v, sem).wait()
            pltpu.sync_copy(z_hbm.at[pl.ds(base, _CH)], z_v)

            def row_body(r, carry):
                a = list(carry)
                for cc in range(_D // 16):
                    sl = pl.ds(cc * 16, 16)
                    q = rows_v[r, sl]
                    z = z_v[r, sl]
                    t = q - z
                    rows_v[r, sl] = z + t
                    a[cc % 8] = a[cc % 8] + t * t
                return tuple(a)

            accs = lax.fori_loop(0, _CH, row_body, accs)
            pltpu.sync_copy(rows_v, out_hbm.at[pl.ds(base, _CH)])
        a = list(accs)
        acc_v[...] = ((a[0] + a[1]) + (a[2] + a[3])) + ((a[4] + a[5]) + (a[6] + a[7]))
        pltpu.sync_copy(acc_v, lp_hbm.at[wid])

    return body(mapped, idx2, z_e2)


def kernel(z_e, codebook_pca, W, b):
    mapped, msq_col, mt = _mapped_call(codebook_pca, W, b[None, :])
    msq_row = msq_col.reshape(1, _K)
    zw = z_e.reshape(_G, _WN, _D).transpose(1, 0, 2)
    zsq3 = jnp.sum(z_e ** 2, axis=1).reshape(_G, _WN).T[..., None]
    idxs = _assign_call(zw, zsq3, mt, msq_row)
    idx2 = idxs.reshape(_NW_SC * _NCH, _CH)
    z_q_st, lp = _sc_gather_st_loss(mapped, idx2, z_e)
    vq = jnp.sum(lp) * (1.0 / (_N * _D))
    loss = 0.75 * vq + 0.25 * vq
    return (z_q_st, loss)
